# Initial kernel scaffold; baseline (speedup 1.0000x reference)
import jax
import jax.numpy as jnp
from jax.experimental import pallas as pl


def kernel(xytp, features, pe1_w, pe1_b, bn_g, bn_b, pe2_w, pe2_b, lt_w, lt_b, ln_g, ln_b):
    raise NotImplementedError("write your pallas kernel here")


# speedup vs baseline: 9.1428x; 9.1428x over previous
import functools

import jax
import jax.numpy as jnp
from jax import lax
from jax.experimental import pallas as pl
from jax.experimental.pallas import tpu as pltpu
from jax.experimental.pallas import tpu_sc as plsc

B, N, K = 4, 4096, 16
CIN, COUT = 32, 32
SCALE = 32.0 ** 0.5
TQ = 256
NQ = N // TQ
DT = 128
_HI = jax.lax.Precision.HIGHEST


def _knn_body(xq_ref, xk_ref, feat_ref, ltw_ref, ltb_ref,
              idx_ref, tab_ref, varphi_ref, acc_ref):
    b = pl.program_id(0)
    q = pl.program_id(1)
    xq = xq_ref[0]
    xk = xk_ref[0]
    q3 = xq[:, 0:3]
    k3 = xk[:, 0:3]
    sqq = jnp.sum(q3 * q3, axis=1)
    sqk = jnp.sum(k3 * k3, axis=1)
    dots = lax.dot_general(q3, k3, (((1,), (1,)), ((), ())),
                           preferred_element_type=jnp.float32)
    d2 = sqq[:, None] + sqk[None, :] - 2.0 * dots
    iota = lax.broadcasted_iota(jnp.int32, (TQ, N), 1)
    inf = jnp.float32(jnp.inf)
    for k in range(K):
        m = jnp.min(d2, axis=1, keepdims=True)
        am = jnp.min(jnp.where(d2 == m, iota, N), axis=1)
        idx_ref[:, k] = am + b * N
        d2 = jnp.where(iota == am[:, None], inf, d2)

    @pl.when((b == 0) & (q == 0))
    def _():
        acc_ref[...] = jnp.zeros_like(acc_ref)

    sel = jnp.where(d2 == inf, 1.0, 0.0)
    zq = jnp.zeros((TQ, 12), jnp.float32)
    zk = jnp.zeros((N, 12), jnp.float32)
    xq_p = jnp.concatenate([xq, zq], axis=1)
    xk_p = jnp.concatenate([xk, zk], axis=1)

    def dot_t(a_, b_):
        return lax.dot_general(a_, b_, (((0,), (0,)), ((), ())),
                               preferred_element_type=jnp.float32,
                               precision=_HI)

    s = lax.dot_general(sel, xk_p, (((1,), (0,)), ((), ())),
                        preferred_element_type=jnp.float32,
                        precision=_HI)
    a_qq = dot_t(xq_p, xq_p)
    cr = dot_t(xq_p, s)
    crt = dot_t(s, xq_p)
    c = jnp.sum(sel, axis=0)
    w = dot_t(xk_p, c[:, None] * xk_p)
    m2blk = float(K) * a_qq - cr - crt + w
    m1blk = float(K) * jnp.sum(xq_p, axis=0) - jnp.sum(s, axis=0)
    acc_ref[0:16, :] += m2blk
    acc_ref[16:17, :] += m1blk[None, :]

    lt = lax.dot_general(feat_ref[0], ltw_ref[...], (((1,), (1,)), ((), ())),
                         preferred_element_type=jnp.float32,
                         precision=_HI) + ltb_ref[0:1, :]
    tab_ref[:, 0:4] = xq
    tab_ref[:, 4:16] = zq
    tab_ref[:, 16:80] = lt[:, 32:96]
    tab_ref[:, 80:128] = jnp.zeros((TQ, 48), jnp.float32)
    varphi_ref[...] = lt[:, 0:32]


def _knn_call(xytp, features, lt_w, ltb_pad):
    return pl.pallas_call(
        _knn_body,
        grid=(B, NQ),
        in_specs=[
            pl.BlockSpec((1, TQ, 4), lambda b, q: (b, q, 0)),
            pl.BlockSpec((1, N, 4), lambda b, q: (b, 0, 0)),
            pl.BlockSpec((1, TQ, CIN), lambda b, q: (b, q, 0)),
            pl.BlockSpec((3 * COUT, CIN), lambda b, q: (0, 0)),
            pl.BlockSpec((8, 3 * COUT), lambda b, q: (0, 0)),
        ],
        out_specs=[
            pl.BlockSpec((TQ, K), lambda b, q: (b * NQ + q, 0)),
            pl.BlockSpec((TQ, DT), lambda b, q: (b * NQ + q, 0)),
            pl.BlockSpec((TQ, COUT), lambda b, q: (b * NQ + q, 0)),
            pl.BlockSpec((24, 16), lambda b, q: (0, 0)),
        ],
        out_shape=[
            jax.ShapeDtypeStruct((B * N, K), jnp.int32),
            jax.ShapeDtypeStruct((B * N, DT), jnp.float32),
            jax.ShapeDtypeStruct((B * N, COUT), jnp.float32),
            jax.ShapeDtypeStruct((24, 16), jnp.float32),
        ],
    )(xytp, xytp, features, lt_w, ltb_pad)


_NW = 32
_IPW = (B * N * K) // _NW
_CH = 128
_NCH = _IPW // _CH


def _sc_gather_body(tab_hbm, idx_hbm, out_hbm, idx_v, rows_v, sem):
    wid = lax.axis_index("s") * 2 + lax.axis_index("c")
    base = wid * _IPW

    def chunk(j, carry):
        off = pl.multiple_of(base + j * _CH, _CH)
        pltpu.sync_copy(idx_hbm.at[pl.ds(off, _CH)], idx_v)
        pltpu.async_copy(tab_hbm.at[idx_v], rows_v, sem).wait()
        pltpu.sync_copy(rows_v, out_hbm.at[pl.ds(off, _CH)])
        return carry

    lax.fori_loop(0, _NCH, chunk, 0)


def _sc_gather_call(tab, idx_flat):
    mesh = plsc.VectorSubcoreMesh(core_axis_name="c", subcore_axis_name="s")
    k = functools.partial(
        pl.kernel,
        out_type=jax.ShapeDtypeStruct((B * N * K, DT), jnp.float32),
        mesh=mesh,
        scratch_types=[
            pltpu.VMEM((_CH,), jnp.int32),
            pltpu.VMEM((_CH, DT), jnp.float32),
            pltpu.SemaphoreType.DMA,
        ],
    )(_sc_gather_body)
    return k(tab, idx_flat)


def _attn_body(g_ref, tq_ref, varphi_ref, w1p_ref, pe2p_ref, vecs_ref,
               out_ref):
    g = g_ref[...]
    xg = g[:, 0:16]
    q = tq_ref[:, 0:16]
    qb = jnp.broadcast_to(q[:, None, :], (TQ, K, 16)).reshape(TQ * K, 16)
    rel = qb - xg
    h = lax.dot_general(rel, w1p_ref[...], (((1,), (0,)), ((), ())),
                        preferred_element_type=jnp.float32,
                        precision=_HI) + vecs_ref[0:1, 0:16]
    h = jnp.maximum(h, 0.0)
    delta = lax.dot_general(h, pe2p_ref[...], (((1,), (0,)), ((), ())),
                            preferred_element_type=jnp.float32,
                            precision=_HI) + vecs_ref[1:2, 0:32]
    deltak = delta.reshape(TQ, K, COUT)
    psi = g[:, 16:48].reshape(TQ, K, COUT)
    alpha = g[:, 48:80].reshape(TQ, K, COUT)
    pre = varphi_ref[...][:, None, :] - psi + deltak
    m = jnp.mean(pre, axis=2, keepdims=True)
    v = jnp.mean((pre - m) ** 2, axis=2, keepdims=True)
    ln_g = vecs_ref[2:3, 0:32].reshape(1, 1, COUT)
    ln_b = vecs_ref[3:4, 0:32].reshape(1, 1, COUT)
    ln = (pre - m) / jnp.sqrt(v + 1e-5) * ln_g + ln_b
    z = ln * (1.0 / SCALE)
    zmax = jnp.max(z, axis=1, keepdims=True)
    e = jnp.exp(z - zmax)
    s = e / jnp.sum(e, axis=1, keepdims=True)
    out_ref[...] = jnp.sum(s * (alpha + deltak), axis=1)


def _attn_call(g, tab, varphi, w1p, pe2p, vecs):
    return pl.pallas_call(
        _attn_body,
        grid=(B * NQ,),
        in_specs=[
            pl.BlockSpec((TQ * K, DT), lambda i: (i, 0)),
            pl.BlockSpec((TQ, DT), lambda i: (i, 0)),
            pl.BlockSpec((TQ, COUT), lambda i: (i, 0)),
            pl.BlockSpec((16, 16), lambda i: (0, 0)),
            pl.BlockSpec((16, COUT), lambda i: (0, 0)),
            pl.BlockSpec((8, 128), lambda i: (0, 0)),
        ],
        out_specs=pl.BlockSpec((TQ, COUT), lambda i: (i, 0)),
        out_shape=jax.ShapeDtypeStruct((B * N, COUT), jnp.float32),
    )(g, tab, varphi, w1p, pe2p, vecs)


def kernel(xytp, features, pe1_w, pe1_b, bn_g, bn_b, pe2_w, pe2_b,
           lt_w, lt_b, ln_g, ln_b):
    ltb_pad = jnp.zeros((8, 3 * COUT), jnp.float32).at[0, :].set(lt_b)
    idx, tab, varphi, acc = _knn_call(xytp, features, lt_w, ltb_pad)
    g = _sc_gather_call(tab, idx.reshape(B * N * K))

    cnt = float(B * N * K)
    m2 = acc[0:4, 0:4] / cnt
    m1 = acc[16, 0:4] / cnt
    mu_h = m1 @ pe1_w.T + pe1_b
    ex2 = (jnp.sum((pe1_w @ m2) * pe1_w, axis=1)
           + 2.0 * pe1_b * (pe1_w @ m1) + pe1_b ** 2)
    var_h = ex2 - mu_h ** 2
    ge = bn_g / jnp.sqrt(var_h + 1e-5)
    w1 = pe1_w * ge[:, None]
    b1 = (pe1_b - mu_h) * ge + bn_b
    w1p = jnp.zeros((16, 16), jnp.float32).at[0:4, 0:4].set(w1.T)
    pe2p = jnp.zeros((16, COUT), jnp.float32).at[0:4, :].set(pe2_w.T)
    vecs = (jnp.zeros((8, 128), jnp.float32)
            .at[0, 0:4].set(b1)
            .at[1, 0:32].set(pe2_b)
            .at[2, 0:32].set(ln_g)
            .at[3, 0:32].set(ln_b))
    out = _attn_call(g, tab, varphi, w1p, pe2p, vecs)
    return out.reshape(B, N, COUT)
